# Initial kernel scaffold; baseline (speedup 1.0000x reference)
import jax
import jax.numpy as jnp
from jax.experimental import pallas as pl


def kernel(obs, action, mem_obs, mem_action, mem_Q):
    raise NotImplementedError("write your pallas kernel here")


# speedup vs baseline: 1.4630x; 1.4630x over previous
import jax
import jax.numpy as jnp
from jax import lax
from jax.experimental import pallas as pl
from jax.experimental.pallas import tpu as pltpu

B, OD, AD = 256, 48, 16
D = OD + AD
M = 250000
K = 10
W = 2048
T = (M + W - 1) // W
AUGPAD = 128
AUG = AUGPAD + W

_DN = (((1,), (1,)), ((), ()))


def _body(q_ref, mo_ref, ma_ref, mq_ref, out_ref, bd_ref, bq_ref):
    t = pl.program_id(0)

    @pl.when(t == 0)
    def _init():
        bd_ref[...] = jnp.full((B, AUGPAD), jnp.inf, jnp.float32)
        bq_ref[...] = jnp.zeros((B, AUGPAD), jnp.float32)

    q = q_ref[...]
    cat = jnp.concatenate([mo_ref[...], ma_ref[...]], axis=1)

    dot = lax.dot_general(q, cat, _DN,
                          preferred_element_type=jnp.float32)
    catT = jnp.transpose(cat)
    m2 = jnp.sum(catT * catT, axis=0, keepdims=True)
    qT = jnp.transpose(q)
    q2 = jnp.transpose(jnp.sum(qT * qT, axis=0, keepdims=True))
    s = (q2 - 2.0 * dot) + m2

    iota1 = lax.broadcasted_iota(jnp.int32, (1, W), 1)
    s = jnp.where(t * W + iota1 < M, s, jnp.inf)

    bq_old = bq_ref[...]
    aug = jnp.concatenate([bd_ref[...], s], axis=1)
    ia = lax.broadcasted_iota(jnp.int32, (B, AUG), 1)
    mqr = mq_ref[...]

    dcols, qcols = [], []
    for _ in range(K):
        mn = jnp.min(aug, axis=1, keepdims=True)
        c = jnp.min(jnp.where(aug == mn, ia, 2 ** 30), axis=1,
                    keepdims=True)
        sel = ia == c
        qv = jnp.sum(jnp.where(sel[:, :AUGPAD], bq_old, 0.0), axis=1,
                     keepdims=True)
        qv = qv + jnp.sum(jnp.where(sel[:, AUGPAD:], mqr, 0.0), axis=1,
                          keepdims=True)
        dcols.append(mn)
        qcols.append(qv)
        aug = jnp.where(sel, jnp.inf, aug)

    newd = jnp.concatenate(
        dcols + [jnp.full((B, AUGPAD - K), jnp.inf, jnp.float32)], axis=1)
    newq = jnp.concatenate(
        qcols + [jnp.zeros((B, AUGPAD - K), jnp.float32)], axis=1)
    bd_ref[...] = newd
    bq_ref[...] = newq

    @pl.when(t == T - 1)
    def _fin():
        lm = lax.broadcasted_iota(jnp.int32, (B, AUGPAD), 1) < K
        dm = jnp.where(lm, newd, -jnp.inf)
        mx = jnp.max(dm, axis=1, keepdims=True)
        e = jnp.exp(dm - mx)
        out_ref[...] = (jnp.sum(e * newq, axis=1, keepdims=True)
                        / jnp.sum(e, axis=1, keepdims=True))


def kernel(obs, action, mem_obs, mem_action, mem_Q):
    q = jnp.concatenate([obs, action], axis=1)
    mq = mem_Q.reshape(1, M)
    out = pl.pallas_call(
        _body,
        grid=(T,),
        in_specs=[
            pl.BlockSpec((B, D), lambda t: (0, 0)),
            pl.BlockSpec((W, OD), lambda t: (t, 0)),
            pl.BlockSpec((W, AD), lambda t: (t, 0)),
            pl.BlockSpec((1, W), lambda t: (0, t)),
        ],
        out_specs=pl.BlockSpec((B, 1), lambda t: (0, 0)),
        out_shape=jax.ShapeDtypeStruct((B, 1), jnp.float32),
        scratch_shapes=[
            pltpu.VMEM((B, AUGPAD), jnp.float32),
            pltpu.VMEM((B, AUGPAD), jnp.float32),
        ],
        compiler_params=pltpu.CompilerParams(
            dimension_semantics=("arbitrary",)),
    )(q, mem_obs, mem_action, mq)
    return out[:, 0]


# speedup vs baseline: 1.6106x; 1.1009x over previous
import functools

import jax
import jax.numpy as jnp
from jax import lax
from jax.experimental import pallas as pl
from jax.experimental.pallas import tpu as pltpu
from jax.experimental.pallas import tpu_sc as plsc

B, OD, AD = 256, 48, 16
D = OD + AD
M = 250000
K = 10
W = 2048
T = (M + W - 1) // W
AUGPAD = 128
AUG = AUGPAD + W

NW = 32
QPW = B // NW

_DN = (((1,), (1,)), ((), ()))


def _tc_body(q_ref, mo_ref, ma_ref, bdo_ref, bio_ref, bd_ref, bi_ref):
    t = pl.program_id(0)

    @pl.when(t == 0)
    def _init():
        bd_ref[...] = jnp.full((B, AUGPAD), jnp.inf, jnp.float32)
        bi_ref[...] = jnp.zeros((B, AUGPAD), jnp.int32)

    q = q_ref[...]
    cat = jnp.concatenate([mo_ref[...], ma_ref[...]], axis=1)

    dot = lax.dot_general(q, cat, _DN,
                          preferred_element_type=jnp.float32)
    catT = jnp.transpose(cat)
    m2 = jnp.sum(catT * catT, axis=0, keepdims=True)
    qT = jnp.transpose(q)
    q2 = jnp.transpose(jnp.sum(qT * qT, axis=0, keepdims=True))
    s = (q2 - 2.0 * dot) + m2

    iota1 = lax.broadcasted_iota(jnp.int32, (1, W), 1)
    s = jnp.where(t * W + iota1 < M, s, jnp.inf)

    bi_old = bi_ref[...]
    aug = jnp.concatenate([bd_ref[...], s], axis=1)
    ia = lax.broadcasted_iota(jnp.int32, (B, AUG), 1)

    dcols, icols = [], []
    for _ in range(K):
        mn = jnp.min(aug, axis=1, keepdims=True)
        c = jnp.min(jnp.where(aug == mn, ia, 2 ** 30), axis=1,
                    keepdims=True)
        sel = ia == c
        gi_old = jnp.sum(jnp.where(sel[:, :AUGPAD], bi_old, 0), axis=1,
                         keepdims=True)
        gi = jnp.where(c < AUGPAD, gi_old, c - AUGPAD + t * W)
        dcols.append(mn)
        icols.append(gi)
        aug = jnp.where(sel, jnp.inf, aug)

    newd = jnp.concatenate(
        dcols + [jnp.full((B, AUGPAD - K), jnp.inf, jnp.float32)], axis=1)
    newi = jnp.concatenate(
        icols + [jnp.zeros((B, AUGPAD - K), jnp.int32)], axis=1)
    bd_ref[...] = newd
    bi_ref[...] = newi

    @pl.when(t == T - 1)
    def _fin():
        bdo_ref[...] = jnp.transpose(newd)
        bio_ref[...] = jnp.transpose(newi)


def _tc_topk(q, mem_obs, mem_action):
    return pl.pallas_call(
        _tc_body,
        grid=(T,),
        in_specs=[
            pl.BlockSpec((B, D), lambda t: (0, 0)),
            pl.BlockSpec((W, OD), lambda t: (t, 0)),
            pl.BlockSpec((W, AD), lambda t: (t, 0)),
        ],
        out_specs=[
            pl.BlockSpec((AUGPAD, B), lambda t: (0, 0)),
            pl.BlockSpec((AUGPAD, B), lambda t: (0, 0)),
        ],
        out_shape=[
            jax.ShapeDtypeStruct((AUGPAD, B), jnp.float32),
            jax.ShapeDtypeStruct((AUGPAD, B), jnp.int32),
        ],
        scratch_shapes=[
            pltpu.VMEM((B, AUGPAD), jnp.float32),
            pltpu.VMEM((B, AUGPAD), jnp.int32),
        ],
        compiler_params=pltpu.CompilerParams(
            dimension_semantics=("arbitrary",)),
    )(q, mem_obs, mem_action)


@functools.partial(
    pl.kernel,
    mesh=plsc.VectorSubcoreMesh(core_axis_name="c", subcore_axis_name="s"),
    out_type=jax.ShapeDtypeStruct((B,), jnp.float32),
    scratch_types=[
        pltpu.VMEM((16, 16), jnp.float32),
        pltpu.VMEM((16, 16), jnp.int32),
        pltpu.VMEM((16, 16), jnp.float32),
        pltpu.VMEM((16,), jnp.float32),
        pltpu.SemaphoreType.DMA,
    ],
)
def _sc_combine(bd_hbm, bi_hbm, mq_hbm, out_hbm, bd_v, bi_v, qs_v, res_v,
                sem):
    nc = 2
    wid = lax.axis_index("s") * nc + lax.axis_index("c")

    @pl.when(wid < 16)
    def _():
        base = wid * 16
        loads = []
        for k in range(K):
            loads.append(pltpu.async_copy(
                bd_hbm.at[pl.ds(k * B + base, 16)], bd_v.at[k, :], sem))
            loads.append(pltpu.async_copy(
                bi_hbm.at[pl.ds(k * B + base, 16)], bi_v.at[k, :], sem))
        for cp in loads:
            cp.wait()
        copies = [
            pltpu.async_copy(mq_hbm.at[bi_v[k, :]], qs_v.at[k, :], sem)
            for k in range(K)
        ]
        for cp in copies:
            cp.wait()
        dvs = [bd_v[k, :] for k in range(K)]
        mx = dvs[0]
        for k in range(1, K):
            mx = jnp.maximum(mx, dvs[k])
        num = jnp.zeros((16,), jnp.float32)
        den = jnp.zeros((16,), jnp.float32)
        for k in range(K):
            e = jnp.exp(dvs[k] - mx)
            num = num + e * qs_v[k, :]
            den = den + e
        res_v[...] = num / den
        pltpu.sync_copy(res_v, out_hbm.at[pl.ds(base, 16)])


def kernel(obs, action, mem_obs, mem_action, mem_Q):
    q = jnp.concatenate([obs, action], axis=1)
    bd, bi = _tc_topk(q, mem_obs, mem_action)
    return _sc_combine(bd.reshape(AUGPAD * B), bi.reshape(AUGPAD * B),
                       mem_Q.reshape(M))


# speedup vs baseline: 2.6270x; 1.6310x over previous
import functools

import jax
import jax.numpy as jnp
from jax import lax
from jax.experimental import pallas as pl
from jax.experimental.pallas import tpu as pltpu
from jax.experimental.pallas import tpu_sc as plsc

B, OD, AD = 256, 48, 16
D = OD + AD
M = 250000
K = 10
W = 2048
T = (M + W - 1) // W
AUGPAD = 128
AUG = AUGPAD + W

NW = 32
QPW = B // NW

_DN = (((1,), (1,)), ((), ()))


def _tc_body(q_ref, mo_ref, ma_ref, bdo_ref, bio_ref, bd_ref, bi_ref, s_ref):
    t = pl.program_id(0)

    @pl.when(t == 0)
    def _init():
        bd_ref[...] = jnp.full((B, AUGPAD), jnp.inf, jnp.float32)
        bi_ref[...] = jnp.zeros((B, AUGPAD), jnp.int32)

    q = q_ref[...]
    cat = jnp.concatenate([mo_ref[...], ma_ref[...]], axis=1)

    dot = lax.dot_general(q, cat, _DN,
                          preferred_element_type=jnp.float32)
    catT = jnp.transpose(cat)
    m2 = jnp.sum(catT * catT, axis=0, keepdims=True)
    qT = jnp.transpose(q)
    q2 = jnp.transpose(jnp.sum(qT * qT, axis=0, keepdims=True))
    s = (q2 - 2.0 * dot) + m2

    iota1 = lax.broadcasted_iota(jnp.int32, (1, W), 1)
    s_ref[...] = jnp.where(t * W + iota1 < M, s, jnp.inf)

    ia = lax.broadcasted_iota(jnp.int32, (B, W), 1)
    lane = lax.broadcasted_iota(jnp.int32, (B, AUGPAD), 1)

    def _round(_):
        sv = s_ref[...]
        mn = jnp.min(sv, axis=1, keepdims=True)
        bd = bd_ref[...]
        imp = mn < bd[:, K - 1:K]
        go = jnp.any(imp)

        @pl.when(go)
        def _insert():
            c = jnp.min(jnp.where(sv == mn, ia, 2 ** 30), axis=1,
                        keepdims=True)
            s_ref[...] = jnp.where(ia == c, jnp.inf, sv)
            gi = c + t * W
            bi = bi_ref[...]
            bd_sh = jnp.concatenate(
                [jnp.full((B, 1), -jnp.inf, jnp.float32), bd[:, :AUGPAD - 1]],
                axis=1)
            bi_sh = jnp.concatenate(
                [jnp.zeros((B, 1), jnp.int32), bi[:, :AUGPAD - 1]], axis=1)
            geq = bd > mn
            geq_s = bd_sh > mn
            bd_new = jnp.where(geq, jnp.where(geq_s, bd_sh, mn), bd)
            bi_new = jnp.where(geq, jnp.where(geq_s, bi_sh, gi), bi)
            bd_ref[...] = jnp.where(imp, bd_new, bd)
            bi_ref[...] = jnp.where(imp, bi_new, bi)

        return jnp.where(go, 1, 0)

    lax.while_loop(lambda go: go != 0, _round, 1)

    @pl.when(t == T - 1)
    def _fin():
        bdo_ref[...] = jnp.transpose(bd_ref[...])
        bio_ref[...] = jnp.transpose(bi_ref[...])


def _tc_topk(q, mem_obs, mem_action):
    return pl.pallas_call(
        _tc_body,
        grid=(T,),
        in_specs=[
            pl.BlockSpec((B, D), lambda t: (0, 0)),
            pl.BlockSpec((W, OD), lambda t: (t, 0)),
            pl.BlockSpec((W, AD), lambda t: (t, 0)),
        ],
        out_specs=[
            pl.BlockSpec((AUGPAD, B), lambda t: (0, 0)),
            pl.BlockSpec((AUGPAD, B), lambda t: (0, 0)),
        ],
        out_shape=[
            jax.ShapeDtypeStruct((AUGPAD, B), jnp.float32),
            jax.ShapeDtypeStruct((AUGPAD, B), jnp.int32),
        ],
        scratch_shapes=[
            pltpu.VMEM((B, AUGPAD), jnp.float32),
            pltpu.VMEM((B, AUGPAD), jnp.int32),
            pltpu.VMEM((B, W), jnp.float32),
        ],
        compiler_params=pltpu.CompilerParams(
            dimension_semantics=("arbitrary",)),
    )(q, mem_obs, mem_action)


def _sc_combine(bd_flat, bi_flat, mq_flat):
    fn = functools.partial(
        pl.kernel,
        mesh=plsc.VectorSubcoreMesh(core_axis_name="c", subcore_axis_name="s"),
        out_type=jax.ShapeDtypeStruct((B,), jnp.float32),
        scratch_types=[
            pltpu.VMEM((16, 16), jnp.float32),
            pltpu.VMEM((16, 16), jnp.int32),
            pltpu.VMEM((16, 16), jnp.float32),
            pltpu.VMEM((16,), jnp.float32),
            pltpu.SemaphoreType.DMA,
        ],
    )(_sc_body)
    return fn(bd_flat, bi_flat, mq_flat)


def _sc_body(bd_hbm, bi_hbm, mq_hbm, out_hbm, bd_v, bi_v, qs_v, res_v,
             sem):
    nc = 2
    wid = lax.axis_index("s") * nc + lax.axis_index("c")

    @pl.when(wid < 16)
    def _():
        base = wid * 16
        loads = []
        for k in range(K):
            loads.append(pltpu.async_copy(
                bd_hbm.at[pl.ds(k * B + base, 16)], bd_v.at[k, :], sem))
            loads.append(pltpu.async_copy(
                bi_hbm.at[pl.ds(k * B + base, 16)], bi_v.at[k, :], sem))
        for cp in loads:
            cp.wait()
        copies = [
            pltpu.async_copy(mq_hbm.at[bi_v[k, :]], qs_v.at[k, :], sem)
            for k in range(K)
        ]
        for cp in copies:
            cp.wait()
        dvs = [bd_v[k, :] for k in range(K)]
        mx = dvs[0]
        for k in range(1, K):
            mx = jnp.maximum(mx, dvs[k])
        num = jnp.zeros((16,), jnp.float32)
        den = jnp.zeros((16,), jnp.float32)
        for k in range(K):
            e = jnp.exp(dvs[k] - mx)
            num = num + e * qs_v[k, :]
            den = den + e
        res_v[...] = num / den
        pltpu.sync_copy(res_v, out_hbm.at[pl.ds(base, 16)])


def kernel(obs, action, mem_obs, mem_action, mem_Q):
    q = jnp.concatenate([obs, action], axis=1)
    bd, bi = _tc_topk(q, mem_obs, mem_action)
    return _sc_combine(bd.reshape(AUGPAD * B), bi.reshape(AUGPAD * B),
                       mem_Q.reshape(M))


# speedup vs baseline: 2.7160x; 1.0339x over previous
import functools

import jax
import jax.numpy as jnp
from jax import lax
from jax.experimental import pallas as pl
from jax.experimental.pallas import tpu as pltpu
from jax.experimental.pallas import tpu_sc as plsc

B, OD, AD = 256, 48, 16
D = OD + AD
M = 250000
K = 10
W = 2048
T = (M + W - 1) // W
AUGPAD = 128
AUG = AUGPAD + W

NW = 32
QPW = B // NW

_DN = (((1,), (1,)), ((), ()))


def _tc_body(q_ref, mo_ref, ma_ref, bdo_ref, bio_ref, bd_ref, bi_ref, s_ref,
             f_ref):
    t = pl.program_id(0)

    @pl.when(t == 0)
    def _init():
        bd_ref[...] = jnp.full((B, AUGPAD), jnp.inf, jnp.float32)
        bi_ref[...] = jnp.zeros((B, AUGPAD), jnp.int32)

    q = q_ref[...]
    cat = jnp.concatenate([mo_ref[...], ma_ref[...]], axis=1)

    dot = lax.dot_general(q, cat, _DN,
                          preferred_element_type=jnp.float32)
    catT = jnp.transpose(cat)
    m2 = jnp.sum(catT * catT, axis=0, keepdims=True)
    qT = jnp.transpose(q)
    q2 = jnp.transpose(jnp.sum(qT * qT, axis=0, keepdims=True))
    s = (q2 - 2.0 * dot) + m2

    iota1 = lax.broadcasted_iota(jnp.int32, (1, W), 1)
    s = jnp.where(t * W + iota1 < M, s, jnp.inf)
    s_ref[...] = s

    f = s[:, 0:128]
    for g in range(1, W // 128):
        f = jnp.minimum(f, s[:, g * 128:(g + 1) * 128])
    f_ref[...] = f

    ia = lax.broadcasted_iota(jnp.int32, (B, W), 1)

    def _round(_):
        mn = jnp.min(f_ref[...], axis=1, keepdims=True)
        bd = bd_ref[...]
        imp = mn < bd[:, K - 1:K]
        go = jnp.any(imp)

        @pl.when(go)
        def _insert():
            sv = s_ref[...]
            c = jnp.min(jnp.where(sv == mn, ia, 2 ** 30), axis=1,
                        keepdims=True)
            masked = jnp.where(ia == c, jnp.inf, sv)
            s_ref[...] = masked
            nf = masked[:, 0:128]
            for g in range(1, W // 128):
                nf = jnp.minimum(nf, masked[:, g * 128:(g + 1) * 128])
            f_ref[...] = nf
            gi = c + t * W
            bi = bi_ref[...]
            bd_sh = jnp.concatenate(
                [jnp.full((B, 1), -jnp.inf, jnp.float32), bd[:, :AUGPAD - 1]],
                axis=1)
            bi_sh = jnp.concatenate(
                [jnp.zeros((B, 1), jnp.int32), bi[:, :AUGPAD - 1]], axis=1)
            geq = bd > mn
            geq_s = bd_sh > mn
            bd_new = jnp.where(geq, jnp.where(geq_s, bd_sh, mn), bd)
            bi_new = jnp.where(geq, jnp.where(geq_s, bi_sh, gi), bi)
            bd_ref[...] = jnp.where(imp, bd_new, bd)
            bi_ref[...] = jnp.where(imp, bi_new, bi)

        return jnp.where(go, 1, 0)

    lax.while_loop(lambda go: go != 0, _round, 1)

    @pl.when(t == T - 1)
    def _fin():
        bdo_ref[...] = jnp.transpose(bd_ref[...])
        bio_ref[...] = jnp.transpose(bi_ref[...])


def _tc_topk(q, mem_obs, mem_action):
    return pl.pallas_call(
        _tc_body,
        grid=(T,),
        in_specs=[
            pl.BlockSpec((B, D), lambda t: (0, 0)),
            pl.BlockSpec((W, OD), lambda t: (t, 0)),
            pl.BlockSpec((W, AD), lambda t: (t, 0)),
        ],
        out_specs=[
            pl.BlockSpec((AUGPAD, B), lambda t: (0, 0)),
            pl.BlockSpec((AUGPAD, B), lambda t: (0, 0)),
        ],
        out_shape=[
            jax.ShapeDtypeStruct((AUGPAD, B), jnp.float32),
            jax.ShapeDtypeStruct((AUGPAD, B), jnp.int32),
        ],
        scratch_shapes=[
            pltpu.VMEM((B, AUGPAD), jnp.float32),
            pltpu.VMEM((B, AUGPAD), jnp.int32),
            pltpu.VMEM((B, W), jnp.float32),
            pltpu.VMEM((B, 128), jnp.float32),
        ],
        compiler_params=pltpu.CompilerParams(
            dimension_semantics=("arbitrary",)),
    )(q, mem_obs, mem_action)


def _sc_combine(bd_flat, bi_flat, mq_flat):
    fn = functools.partial(
        pl.kernel,
        mesh=plsc.VectorSubcoreMesh(core_axis_name="c", subcore_axis_name="s"),
        out_type=jax.ShapeDtypeStruct((B,), jnp.float32),
        scratch_types=[
            pltpu.VMEM((16, 16), jnp.float32),
            pltpu.VMEM((16, 16), jnp.int32),
            pltpu.VMEM((16, 16), jnp.float32),
            pltpu.VMEM((16,), jnp.float32),
            pltpu.SemaphoreType.DMA,
        ],
    )(_sc_body)
    return fn(bd_flat, bi_flat, mq_flat)


def _sc_body(bd_hbm, bi_hbm, mq_hbm, out_hbm, bd_v, bi_v, qs_v, res_v,
             sem):
    nc = 2
    wid = lax.axis_index("s") * nc + lax.axis_index("c")

    @pl.when(wid < 16)
    def _():
        base = wid * 16
        loads = []
        for k in range(K):
            loads.append(pltpu.async_copy(
                bd_hbm.at[pl.ds(k * B + base, 16)], bd_v.at[k, :], sem))
            loads.append(pltpu.async_copy(
                bi_hbm.at[pl.ds(k * B + base, 16)], bi_v.at[k, :], sem))
        for cp in loads:
            cp.wait()
        copies = [
            pltpu.async_copy(mq_hbm.at[bi_v[k, :]], qs_v.at[k, :], sem)
            for k in range(K)
        ]
        for cp in copies:
            cp.wait()
        dvs = [bd_v[k, :] for k in range(K)]
        mx = dvs[0]
        for k in range(1, K):
            mx = jnp.maximum(mx, dvs[k])
        num = jnp.zeros((16,), jnp.float32)
        den = jnp.zeros((16,), jnp.float32)
        for k in range(K):
            e = jnp.exp(dvs[k] - mx)
            num = num + e * qs_v[k, :]
            den = den + e
        res_v[...] = num / den
        pltpu.sync_copy(res_v, out_hbm.at[pl.ds(base, 16)])


def kernel(obs, action, mem_obs, mem_action, mem_Q):
    q = jnp.concatenate([obs, action], axis=1)
    bd, bi = _tc_topk(q, mem_obs, mem_action)
    return _sc_combine(bd.reshape(AUGPAD * B), bi.reshape(AUGPAD * B),
                       mem_Q.reshape(M))
